# baseline (speedup 1.0000x reference)
import functools

import jax
import jax.numpy as jnp
from jax import lax
from jax.experimental import pallas as pl
from jax.experimental.pallas import tpu as pltpu
from jax.experimental.pallas import tpu_sc as plsc

N = 10000
K = 16
PAD = 10240
RBLK = 256
NW = 32
ROWS_PER_W = PAD // NW
CHUNK_ROWS = 8
EDGES_PER_CHUNK = CHUNK_ROWS * K
NCHUNK = ROWS_PER_W // CHUNK_ROWS


CW = 128
NCHK = PAD // CW
DEPTH = 4
BIGI = 1 << 30


def _top4(chunk, liota):
    inf = jnp.inf
    v1 = jnp.min(chunk, axis=1, keepdims=True)
    c1 = jnp.min(jnp.where(chunk == v1, liota, BIGI), axis=1, keepdims=True)
    ex = liota == c1
    v2 = jnp.min(jnp.where(ex, inf, chunk), axis=1, keepdims=True)
    c2 = jnp.min(jnp.where((chunk == v2) & ~ex, liota, BIGI),
                 axis=1, keepdims=True)
    ex = ex | (liota == c2)
    v3 = jnp.min(jnp.where(ex, inf, chunk), axis=1, keepdims=True)
    c3 = jnp.min(jnp.where((chunk == v3) & ~ex, liota, BIGI),
                 axis=1, keepdims=True)
    ex = ex | (liota == c3)
    v4 = jnp.min(jnp.where(ex, inf, chunk), axis=1, keepdims=True)
    c4 = jnp.min(jnp.where((chunk == v4) & ~ex, liota, BIGI),
                 axis=1, keepdims=True)
    return (v1, v2, v3, v4), (c1, c2, c3, c4)


def _knn_body(pos_ref, post_ref, idx_ref, d2_ref):
    pos = pos_ref[...]
    sqi = (pos[:, 0:1] * pos[:, 0:1] + pos[:, 1:2] * pos[:, 1:2]) \
        + pos[:, 2:3] * pos[:, 2:3]
    liota = lax.broadcasted_iota(jnp.int32, (RBLK, CW), 1)
    chiota = lax.broadcasted_iota(jnp.int32, (RBLK, NCHK), 1)
    kcol = lax.broadcasted_iota(jnp.int32, (RBLK, K), 1)

    def merge_cache(cache, c, vs, cs):
        mv, mc = cache
        sel = chiota == c
        mv = tuple(jnp.where(sel, v, m) for v, m in zip(vs, mv))
        mc = tuple(jnp.where(sel, cc, m) for cc, m in zip(cs, mc))
        return mv, mc

    def build(c, cache):
        off = pl.multiple_of(c * CW, CW)
        postc = post_ref[:, pl.ds(off, CW)]
        dotc = jnp.dot(pos, postc, preferred_element_type=jnp.float32)
        sqjc = (postc[0:1, :] * postc[0:1, :]
                + postc[1:2, :] * postc[1:2, :]) \
            + postc[2:3, :] * postc[2:3, :]
        colc = liota + off
        d2c = sqi + sqjc - 2.0 * dotc
        d2c = jnp.where(colc >= N, jnp.inf, d2c)
        d2_ref[:, pl.ds(off, CW)] = d2c
        vs, cs = _top4(d2c, liota)
        return merge_cache(cache, c, vs, cs)

    z = jnp.zeros((RBLK, NCHK), jnp.float32)
    zi = jnp.zeros((RBLK, NCHK), jnp.int32)
    cache0 = ((z, z, z, z), (zi, zi, zi, zi))
    mv, mc = lax.fori_loop(0, NCHK, build, cache0)

    def ext(t, carry):
        res, head, mv, mc = carry

        def rebuild(ops):
            res_, head_, mv_, mc_ = ops

            def rb(c, cache):
                off = pl.multiple_of(c * CW, CW)
                chunk = d2_ref[:, pl.ds(off, CW)]
                colc = liota + off
                ex = jnp.zeros((RBLK, CW), jnp.bool_)
                for k in range(K):
                    ex = ex | ((colc == res_[:, k:k + 1])
                               & (jnp.int32(k) < t))
                chunk = jnp.where(ex, jnp.inf, chunk)
                vs, cs = _top4(chunk, liota)
                return merge_cache(cache, c, vs, cs)

            nmv, nmc = lax.fori_loop(0, NCHK, rb, (mv_, mc_))
            return res_, jnp.zeros_like(head_), nmv, nmc

        def headvals(head, mv, mc):
            hv = jnp.where(head == 0, mv[0],
                           jnp.where(head == 1, mv[1],
                                     jnp.where(head == 2, mv[2],
                                               jnp.where(head == 3, mv[3],
                                                         jnp.inf))))
            hc = jnp.where(head == 0, mc[0],
                           jnp.where(head == 1, mc[1],
                                     jnp.where(head == 2, mc[2], mc[3])))
            return hv, chiota * CW + hc

        hv, gc = headvals(head, mv, mc)
        w = jnp.min(hv, axis=1, keepdims=True)
        bad = (head >= DEPTH) & (mv[DEPTH - 1] <= w)
        anybad = jnp.max(bad.astype(jnp.int32)) > 0
        res, head, mv, mc = lax.cond(anybad, rebuild, lambda o: o,
                                     (res, head, mv, mc))
        hv, gc = headvals(head, mv, mc)
        w = jnp.min(hv, axis=1, keepdims=True)
        wg = jnp.min(jnp.where(hv == w, gc, BIGI), axis=1, keepdims=True)
        res = jnp.where(kcol == t, wg, res)
        head = jnp.where(chiota == wg // CW, head + 1, head)
        return res, head, mv, mc

    res0 = jnp.zeros((RBLK, K), jnp.int32)
    head0 = jnp.zeros((RBLK, NCHK), jnp.int32)
    res, _, _, _ = lax.fori_loop(0, K, ext, (res0, head0, mv, mc))
    idx_ref[...] = res


def _knn(pos8, post8):
    return pl.pallas_call(
        _knn_body,
        grid=(PAD // RBLK,),
        in_specs=[
            pl.BlockSpec((RBLK, 8), lambda i: (i, 0)),
            pl.BlockSpec((8, PAD), lambda i: (0, 0)),
        ],
        out_specs=pl.BlockSpec((RBLK, K), lambda i: (i, 0)),
        out_shape=jax.ShapeDtypeStruct((PAD, K), jnp.int32),
        scratch_shapes=[pltpu.VMEM((RBLK, PAD), jnp.float32)],
    )(pos8, post8)


def _mm_body(x_ref, w_ref, b_ref, a_ref, g_ref, *, c_in):
    x = x_ref[...]
    w = w_ref[...]
    wt = w[0:c_in, :]
    wb = w[c_in:2 * c_in, :]
    g_ref[...] = jnp.dot(x, wb, preferred_element_type=jnp.float32)
    a_ref[...] = jnp.dot(x, wt - wb, preferred_element_type=jnp.float32) \
        + b_ref[...]


MMB = 1024


def _mm(xp, w, b2d, c_in, c_out):
    return pl.pallas_call(
        functools.partial(_mm_body, c_in=c_in),
        grid=(PAD // MMB,),
        in_specs=[
            pl.BlockSpec((MMB, c_in), lambda i: (i, 0)),
            pl.BlockSpec((2 * c_in, c_out), lambda i: (0, 0)),
            pl.BlockSpec((1, c_out), lambda i: (0, 0)),
        ],
        out_specs=[pl.BlockSpec((MMB, c_out), lambda i: (i, 0)),
                   pl.BlockSpec((MMB, c_out), lambda i: (i, 0))],
        out_shape=[jax.ShapeDtypeStruct((PAD, c_out), jnp.float32),
                   jax.ShapeDtypeStruct((PAD, c_out), jnp.float32)],
    )(xp, w, b2d)


@functools.cache
def _make_gather_max(c_out):
    nseg = c_out // 16
    mesh = plsc.VectorSubcoreMesh(core_axis_name="c", subcore_axis_name="s")

    @functools.partial(
        pl.kernel, mesh=mesh,
        out_type=jax.ShapeDtypeStruct((PAD, c_out), jnp.float32),
        scratch_types=[
            pltpu.VMEM((EDGES_PER_CHUNK,), jnp.int32),
            pltpu.VMEM((EDGES_PER_CHUNK, c_out), jnp.float32),
            pltpu.VMEM((ROWS_PER_W, c_out), jnp.float32),
            pltpu.VMEM((ROWS_PER_W, c_out), jnp.float32),
            pltpu.SemaphoreType.DMA,
        ],
        compiler_params=pltpu.CompilerParams(use_tc_tiling_on_sc=False),
    )
    def gather_max(idx_hbm, g_hbm, a_hbm, out_hbm,
                   idx_v, rows_v, a_v, out_v, sem):
        wid = lax.axis_index("s") * 2 + lax.axis_index("c")
        base = wid * ROWS_PER_W
        pltpu.sync_copy(a_hbm.at[pl.ds(base, ROWS_PER_W)], a_v)

        def chunk(kk, carry):
            ebase = base * K + kk * EDGES_PER_CHUNK
            pltpu.sync_copy(idx_hbm.at[pl.ds(ebase, EDGES_PER_CHUNK)], idx_v)
            pltpu.async_copy(g_hbm.at[idx_v], rows_v, sem).wait()

            def row(r, c2):
                e0 = r * K
                orow = kk * CHUNK_ROWS + r
                for s in range(nseg):
                    sl = pl.ds(s * 16, 16)
                    acc = rows_v[e0, sl]
                    for j in range(1, K):
                        acc = jnp.maximum(acc, rows_v[e0 + j, sl])
                    out_v[orow, sl] = jnp.maximum(acc + a_v[orow, sl], 0.0)
                return c2

            lax.fori_loop(0, CHUNK_ROWS, row, 0)
            return carry

        lax.fori_loop(0, NCHUNK, chunk, 0)
        pltpu.sync_copy(out_v, out_hbm.at[pl.ds(base, ROWS_PER_W)])

    return gather_max


def kernel(point_coords, point_features, W0, b0, W1, b1, W2, b2):
    pos = point_coords[:, 1:4]
    pos8 = jnp.zeros((PAD, 8), jnp.float32).at[:N, :3].set(pos)
    post8 = pos8.T
    idx_flat = _knn(pos8, post8).reshape(PAD * K)

    xp = jnp.zeros((PAD, point_features.shape[1]), jnp.float32)
    xp = xp.at[:N].set(point_features)
    for w, b in ((W0, b0), (W1, b1), (W2, b2)):
        c_in, c_out = w.shape[0] // 2, w.shape[1]
        a, g = _mm(xp, w, b.reshape(1, c_out), c_in, c_out)
        xp = _make_gather_max(c_out)(idx_flat, g, a)
    return xp[:N]


# speedup vs baseline: 1.8862x; 1.8862x over previous
import functools

import jax
import jax.numpy as jnp
from jax import lax
from jax.experimental import pallas as pl
from jax.experimental.pallas import tpu as pltpu
from jax.experimental.pallas import tpu_sc as plsc

N = 10000
K = 16
PAD = 10240
RBLK = 256
NW = 32
ROWS_PER_W = PAD // NW
CHUNK_ROWS = 8
EDGES_PER_CHUNK = CHUNK_ROWS * K
NCHUNK = ROWS_PER_W // CHUNK_ROWS


CW = 512
NCH = PAD // CW
DEPTH = 6
POOL = NCH * DEPTH
BIGI = 1 << 30


def _knn_body(pos_ref, post_ref, idx_ref, d2_ref):
    pos = pos_ref[...]
    sqi = (pos[:, 0:1] * pos[:, 0:1] + pos[:, 1:2] * pos[:, 1:2]) \
        + pos[:, 2:3] * pos[:, 2:3]
    citer = lax.broadcasted_iota(jnp.int32, (RBLK, CW), 1)
    piota = lax.broadcasted_iota(jnp.int32, (RBLK, 128), 1)
    kcol = lax.broadcasted_iota(jnp.int32, (RBLK, K), 1)

    def build(c, carry):
        pv, pc = carry
        off = pl.multiple_of(c * CW, CW)
        postc = post_ref[:, pl.ds(off, CW)]
        dotc = jnp.dot(pos, postc, preferred_element_type=jnp.float32)
        sqjc = (postc[0:1, :] * postc[0:1, :]
                + postc[1:2, :] * postc[1:2, :]) \
            + postc[2:3, :] * postc[2:3, :]
        colc = citer + off
        d2c = sqi + sqjc - 2.0 * dotc
        d2c = jnp.where(colc >= N, jnp.inf, d2c)
        d2_ref[:, pl.ds(off, CW)] = d2c
        for j in range(DEPTH):
            cm = jnp.min(d2c, axis=1, keepdims=True)
            cam = jnp.min(jnp.where(d2c == cm, colc, BIGI),
                          axis=1, keepdims=True)
            d2c = jnp.where(colc == cam, jnp.inf, d2c)
            lane = c * DEPTH + j
            pv = jnp.where(piota == lane, cm, pv)
            pc = jnp.where(piota == lane, cam, pc)
        return pv, pc

    pv0 = jnp.full((RBLK, 128), jnp.inf, jnp.float32)
    pc0 = jnp.full((RBLK, 128), BIGI, jnp.int32)
    pv, pc = lax.fori_loop(0, NCH, build, (pv0, pc0))

    res = jnp.zeros((RBLK, K), jnp.int32)
    cnt = jnp.zeros((RBLK, 128), jnp.int32)
    for t in range(K):
        cm = jnp.min(pv, axis=1, keepdims=True)
        amc = jnp.min(jnp.where(pv == cm, pc, BIGI), axis=1, keepdims=True)
        res = jnp.where(kcol == t, amc, res)
        pv = jnp.where(pc == amc, jnp.inf, pv)
        cnt = cnt + jnp.where(piota == amc // CW, 1, 0)
    anybad = jnp.max(cnt) >= DEPTH

    def fallback(res_):
        def sel(t, carry):
            res, am_prev = carry

            def scan_chunk(c, mcarry):
                m, am = mcarry
                off = pl.multiple_of(c * CW, CW)
                chunk = d2_ref[:, pl.ds(off, CW)]
                colc = citer + off
                chunk = jnp.where(colc == am_prev, jnp.inf, chunk)
                d2_ref[:, pl.ds(off, CW)] = chunk
                cm = jnp.min(chunk, axis=1, keepdims=True)
                cam = jnp.min(jnp.where(chunk <= cm, colc, PAD),
                              axis=1, keepdims=True)
                take = (cm < m) | ((cm == m) & (cam < am))
                return jnp.where(take, cm, m), jnp.where(take, cam, am)

            m0 = jnp.full((RBLK, 1), jnp.inf, jnp.float32)
            am0 = jnp.full((RBLK, 1), PAD, jnp.int32)
            m, am = lax.fori_loop(0, NCH, scan_chunk, (m0, am0))
            return jnp.where(kcol == t, am, res), am

        res0 = jnp.zeros((RBLK, K), jnp.int32)
        amp0 = jnp.full((RBLK, 1), -1, jnp.int32)
        out, _ = lax.fori_loop(0, K, sel, (res0, amp0))
        return out

    res = lax.cond(anybad, fallback, lambda r: r, res)
    idx_ref[...] = res


def _knn(pos8, post8):
    return pl.pallas_call(
        _knn_body,
        grid=(PAD // RBLK,),
        in_specs=[
            pl.BlockSpec((RBLK, 8), lambda i: (i, 0)),
            pl.BlockSpec((8, PAD), lambda i: (0, 0)),
        ],
        out_specs=pl.BlockSpec((RBLK, K), lambda i: (i, 0)),
        out_shape=jax.ShapeDtypeStruct((PAD, K), jnp.int32),
        scratch_shapes=[pltpu.VMEM((RBLK, PAD), jnp.float32)],
    )(pos8, post8)


def _mm_body(x_ref, w_ref, b_ref, a_ref, g_ref, *, c_in):
    x = x_ref[...]
    w = w_ref[...]
    wt = w[0:c_in, :]
    wb = w[c_in:2 * c_in, :]
    g_ref[...] = jnp.dot(x, wb, preferred_element_type=jnp.float32)
    a_ref[...] = jnp.dot(x, wt - wb, preferred_element_type=jnp.float32) \
        + b_ref[...]


MMB = 1024


def _mm(xp, w, b2d, c_in, c_out):
    return pl.pallas_call(
        functools.partial(_mm_body, c_in=c_in),
        grid=(PAD // MMB,),
        in_specs=[
            pl.BlockSpec((MMB, c_in), lambda i: (i, 0)),
            pl.BlockSpec((2 * c_in, c_out), lambda i: (0, 0)),
            pl.BlockSpec((1, c_out), lambda i: (0, 0)),
        ],
        out_specs=[pl.BlockSpec((MMB, c_out), lambda i: (i, 0)),
                   pl.BlockSpec((MMB, c_out), lambda i: (i, 0))],
        out_shape=[jax.ShapeDtypeStruct((PAD, c_out), jnp.float32),
                   jax.ShapeDtypeStruct((PAD, c_out), jnp.float32)],
    )(xp, w, b2d)


@functools.cache
def _make_gather_max(c_out):
    nseg = c_out // 16
    mesh = plsc.VectorSubcoreMesh(core_axis_name="c", subcore_axis_name="s")

    @functools.partial(
        pl.kernel, mesh=mesh,
        out_type=jax.ShapeDtypeStruct((PAD, c_out), jnp.float32),
        scratch_types=[
            pltpu.VMEM((EDGES_PER_CHUNK,), jnp.int32),
            pltpu.VMEM((EDGES_PER_CHUNK, c_out), jnp.float32),
            pltpu.VMEM((ROWS_PER_W, c_out), jnp.float32),
            pltpu.VMEM((ROWS_PER_W, c_out), jnp.float32),
            pltpu.SemaphoreType.DMA,
        ],
        compiler_params=pltpu.CompilerParams(use_tc_tiling_on_sc=False),
    )
    def gather_max(idx_hbm, g_hbm, a_hbm, out_hbm,
                   idx_v, rows_v, a_v, out_v, sem):
        wid = lax.axis_index("s") * 2 + lax.axis_index("c")
        base = wid * ROWS_PER_W
        pltpu.sync_copy(a_hbm.at[pl.ds(base, ROWS_PER_W)], a_v)

        def chunk(kk, carry):
            ebase = base * K + kk * EDGES_PER_CHUNK
            pltpu.sync_copy(idx_hbm.at[pl.ds(ebase, EDGES_PER_CHUNK)], idx_v)
            pltpu.async_copy(g_hbm.at[idx_v], rows_v, sem).wait()

            def row(r, c2):
                e0 = r * K
                orow = kk * CHUNK_ROWS + r
                for s in range(nseg):
                    sl = pl.ds(s * 16, 16)
                    acc = rows_v[e0, sl]
                    for j in range(1, K):
                        acc = jnp.maximum(acc, rows_v[e0 + j, sl])
                    out_v[orow, sl] = jnp.maximum(acc + a_v[orow, sl], 0.0)
                return c2

            lax.fori_loop(0, CHUNK_ROWS, row, 0)
            return carry

        lax.fori_loop(0, NCHUNK, chunk, 0)
        pltpu.sync_copy(out_v, out_hbm.at[pl.ds(base, ROWS_PER_W)])

    return gather_max


def kernel(point_coords, point_features, W0, b0, W1, b1, W2, b2):
    pos = point_coords[:, 1:4]
    pos8 = jnp.zeros((PAD, 8), jnp.float32).at[:N, :3].set(pos)
    post8 = pos8.T
    idx_flat = _knn(pos8, post8).reshape(PAD * K)

    xp = jnp.zeros((PAD, point_features.shape[1]), jnp.float32)
    xp = xp.at[:N].set(point_features)
    for w, b in ((W0, b0), (W1, b1), (W2, b2)):
        c_in, c_out = w.shape[0] // 2, w.shape[1]
        a, g = _mm(xp, w, b.reshape(1, c_out), c_in, c_out)
        xp = _make_gather_max(c_out)(idx_flat, g, a)
    return xp[:N]


# speedup vs baseline: 2.9956x; 1.5882x over previous
import functools

import jax
import jax.numpy as jnp
from jax import lax
from jax.experimental import pallas as pl
from jax.experimental.pallas import tpu as pltpu
from jax.experimental.pallas import tpu_sc as plsc

N = 10000
K = 16
PAD = 10240
RBLK = 256
NW = 32
ROWS_PER_W = PAD // NW
CHUNK_ROWS = 8
EDGES_PER_CHUNK = CHUNK_ROWS * K
NCHUNK = ROWS_PER_W // CHUNK_ROWS


CW = 640
NCH = PAD // CW
DEPTH = 8
POOL = NCH * DEPTH
BIGI = 1 << 30


def _knn_body(pos_ref, post_ref, idx_ref, d2_ref):
    pos = pos_ref[...]
    sqi = (pos[:, 0:1] * pos[:, 0:1] + pos[:, 1:2] * pos[:, 1:2]) \
        + pos[:, 2:3] * pos[:, 2:3]
    citer = lax.broadcasted_iota(jnp.int32, (RBLK, CW), 1)
    piota = lax.broadcasted_iota(jnp.int32, (RBLK, 128), 1)
    kcol = lax.broadcasted_iota(jnp.int32, (RBLK, K), 1)

    def build(c, carry):
        pv, pc = carry
        off = pl.multiple_of(c * CW, CW)
        postc = post_ref[:, pl.ds(off, CW)]
        dotc = jnp.dot(pos, postc, preferred_element_type=jnp.float32)
        sqjc = (postc[0:1, :] * postc[0:1, :]
                + postc[1:2, :] * postc[1:2, :]) \
            + postc[2:3, :] * postc[2:3, :]
        colc = citer + off
        d2c = sqi + sqjc - 2.0 * dotc
        d2c = jnp.where(colc >= N, jnp.inf, d2c)
        d2_ref[:, pl.ds(off, CW)] = d2c
        for j in range(DEPTH):
            cm = jnp.min(d2c, axis=1, keepdims=True)
            cam = jnp.min(jnp.where(d2c == cm, colc, BIGI),
                          axis=1, keepdims=True)
            d2c = jnp.where(colc == cam, jnp.inf, d2c)
            lane = c * DEPTH + j
            pv = jnp.where(piota == lane, cm, pv)
            pc = jnp.where(piota == lane, cam, pc)
        return pv, pc

    pv0 = jnp.full((RBLK, 128), jnp.inf, jnp.float32)
    pc0 = jnp.full((RBLK, 128), BIGI, jnp.int32)
    pv, pc = lax.fori_loop(0, NCH, build, (pv0, pc0))

    res = jnp.zeros((RBLK, K), jnp.int32)
    cnt = jnp.zeros((RBLK, 128), jnp.int32)
    for t in range(K):
        cm = jnp.min(pv, axis=1, keepdims=True)
        amc = jnp.min(jnp.where(pv == cm, pc, BIGI), axis=1, keepdims=True)
        res = jnp.where(kcol == t, amc, res)
        pv = jnp.where(pc == amc, jnp.inf, pv)
        cnt = cnt + jnp.where(piota == amc // CW, 1, 0)
    anybad = jnp.max(cnt) >= DEPTH

    def fallback(res_):
        def sel(t, carry):
            res, am_prev = carry

            def scan_chunk(c, mcarry):
                m, am = mcarry
                off = pl.multiple_of(c * CW, CW)
                chunk = d2_ref[:, pl.ds(off, CW)]
                colc = citer + off
                chunk = jnp.where(colc == am_prev, jnp.inf, chunk)
                d2_ref[:, pl.ds(off, CW)] = chunk
                cm = jnp.min(chunk, axis=1, keepdims=True)
                cam = jnp.min(jnp.where(chunk <= cm, colc, PAD),
                              axis=1, keepdims=True)
                take = (cm < m) | ((cm == m) & (cam < am))
                return jnp.where(take, cm, m), jnp.where(take, cam, am)

            m0 = jnp.full((RBLK, 1), jnp.inf, jnp.float32)
            am0 = jnp.full((RBLK, 1), PAD, jnp.int32)
            m, am = lax.fori_loop(0, NCH, scan_chunk, (m0, am0))
            return jnp.where(kcol == t, am, res), am

        res0 = jnp.zeros((RBLK, K), jnp.int32)
        amp0 = jnp.full((RBLK, 1), -1, jnp.int32)
        out, _ = lax.fori_loop(0, K, sel, (res0, amp0))
        return out

    res = lax.cond(anybad, fallback, lambda r: r, res)
    idx_ref[...] = res


def _knn(pos8, post8):
    return pl.pallas_call(
        _knn_body,
        grid=(PAD // RBLK,),
        in_specs=[
            pl.BlockSpec((RBLK, 8), lambda i: (i, 0)),
            pl.BlockSpec((8, PAD), lambda i: (0, 0)),
        ],
        out_specs=pl.BlockSpec((RBLK, K), lambda i: (i, 0)),
        out_shape=jax.ShapeDtypeStruct((PAD, K), jnp.int32),
        scratch_shapes=[pltpu.VMEM((RBLK, PAD), jnp.float32)],
    )(pos8, post8)


def _mm_body(x_ref, w_ref, b_ref, a_ref, g_ref, *, c_in):
    x = x_ref[...]
    w = w_ref[...]
    wt = w[0:c_in, :]
    wb = w[c_in:2 * c_in, :]
    g_ref[...] = jnp.dot(x, wb, preferred_element_type=jnp.float32)
    a_ref[...] = jnp.dot(x, wt - wb, preferred_element_type=jnp.float32) \
        + b_ref[...]


MMB = 1024


def _mm(xp, w, b2d, c_in, c_out):
    return pl.pallas_call(
        functools.partial(_mm_body, c_in=c_in),
        grid=(PAD // MMB,),
        in_specs=[
            pl.BlockSpec((MMB, c_in), lambda i: (i, 0)),
            pl.BlockSpec((2 * c_in, c_out), lambda i: (0, 0)),
            pl.BlockSpec((1, c_out), lambda i: (0, 0)),
        ],
        out_specs=[pl.BlockSpec((MMB, c_out), lambda i: (i, 0)),
                   pl.BlockSpec((MMB, c_out), lambda i: (i, 0))],
        out_shape=[jax.ShapeDtypeStruct((PAD, c_out), jnp.float32),
                   jax.ShapeDtypeStruct((PAD, c_out), jnp.float32)],
    )(xp, w, b2d)


@functools.cache
def _make_gather_max(c_out):
    nseg = c_out // 16
    mesh = plsc.VectorSubcoreMesh(core_axis_name="c", subcore_axis_name="s")

    @functools.partial(
        pl.kernel, mesh=mesh,
        out_type=jax.ShapeDtypeStruct((PAD, c_out), jnp.float32),
        scratch_types=[
            pltpu.VMEM((EDGES_PER_CHUNK,), jnp.int32),
            pltpu.VMEM((EDGES_PER_CHUNK, c_out), jnp.float32),
            pltpu.VMEM((ROWS_PER_W, c_out), jnp.float32),
            pltpu.VMEM((ROWS_PER_W, c_out), jnp.float32),
            pltpu.SemaphoreType.DMA,
        ],
        compiler_params=pltpu.CompilerParams(use_tc_tiling_on_sc=False),
    )
    def gather_max(idx_hbm, g_hbm, a_hbm, out_hbm,
                   idx_v, rows_v, a_v, out_v, sem):
        wid = lax.axis_index("s") * 2 + lax.axis_index("c")
        base = wid * ROWS_PER_W
        pltpu.sync_copy(a_hbm.at[pl.ds(base, ROWS_PER_W)], a_v)

        def chunk(kk, carry):
            ebase = base * K + kk * EDGES_PER_CHUNK
            pltpu.sync_copy(idx_hbm.at[pl.ds(ebase, EDGES_PER_CHUNK)], idx_v)
            pltpu.async_copy(g_hbm.at[idx_v], rows_v, sem).wait()

            def row(r, c2):
                e0 = r * K
                orow = kk * CHUNK_ROWS + r
                for s in range(nseg):
                    sl = pl.ds(s * 16, 16)
                    acc = rows_v[e0, sl]
                    for j in range(1, K):
                        acc = jnp.maximum(acc, rows_v[e0 + j, sl])
                    out_v[orow, sl] = jnp.maximum(acc + a_v[orow, sl], 0.0)
                return c2

            lax.fori_loop(0, CHUNK_ROWS, row, 0)
            return carry

        lax.fori_loop(0, NCHUNK, chunk, 0)
        pltpu.sync_copy(out_v, out_hbm.at[pl.ds(base, ROWS_PER_W)])

    return gather_max


def kernel(point_coords, point_features, W0, b0, W1, b1, W2, b2):
    pos = point_coords[:, 1:4]
    pos8 = jnp.zeros((PAD, 8), jnp.float32).at[:N, :3].set(pos)
    post8 = pos8.T
    idx_flat = _knn(pos8, post8).reshape(PAD * K)

    xp = jnp.zeros((PAD, point_features.shape[1]), jnp.float32)
    xp = xp.at[:N].set(point_features)
    for w, b in ((W0, b0), (W1, b1), (W2, b2)):
        c_in, c_out = w.shape[0] // 2, w.shape[1]
        a, g = _mm(xp, w, b.reshape(1, c_out), c_in, c_out)
        xp = _make_gather_max(c_out)(idx_flat, g, a)
    return xp[:N]


# speedup vs baseline: 3.9130x; 1.3062x over previous
import functools

import jax
import jax.numpy as jnp
from jax import lax
from jax.experimental import pallas as pl
from jax.experimental.pallas import tpu as pltpu
from jax.experimental.pallas import tpu_sc as plsc

N = 10000
K = 16
PAD = 10240
RBLK = 512
NW = 32
ROWS_PER_W = PAD // NW
CHUNK_ROWS = 8
EDGES_PER_CHUNK = CHUNK_ROWS * K
NCHUNK = ROWS_PER_W // CHUNK_ROWS


CW = 640
NCH = PAD // CW
DEPTH = 8
POOL = NCH * DEPTH
BIGI = 1 << 30


def _knn_body(pos_ref, post_ref, idx_ref, d2_ref):
    pos = pos_ref[...]
    sqi = (pos[:, 0:1] * pos[:, 0:1] + pos[:, 1:2] * pos[:, 1:2]) \
        + pos[:, 2:3] * pos[:, 2:3]
    citer = lax.broadcasted_iota(jnp.int32, (RBLK, CW), 1)
    piota = lax.broadcasted_iota(jnp.int32, (RBLK, 128), 1)
    kcol = lax.broadcasted_iota(jnp.int32, (RBLK, K), 1)

    def build(c, carry):
        pv, pc = carry
        off = pl.multiple_of(c * CW, CW)
        postc = post_ref[:, pl.ds(off, CW)]
        dotc = jnp.dot(pos, postc, preferred_element_type=jnp.float32)
        sqjc = (postc[0:1, :] * postc[0:1, :]
                + postc[1:2, :] * postc[1:2, :]) \
            + postc[2:3, :] * postc[2:3, :]
        colc = citer + off
        d2c = sqi + sqjc - 2.0 * dotc
        d2c = jnp.where(colc >= N, jnp.inf, d2c)
        d2_ref[:, pl.ds(off, CW)] = d2c
        for j in range(DEPTH):
            cm = jnp.min(d2c, axis=1, keepdims=True)
            cam = jnp.min(jnp.where(d2c == cm, colc, BIGI),
                          axis=1, keepdims=True)
            d2c = jnp.where(colc == cam, jnp.inf, d2c)
            lane = c * DEPTH + j
            pv = jnp.where(piota == lane, cm, pv)
            pc = jnp.where(piota == lane, cam, pc)
        return pv, pc

    pv0 = jnp.full((RBLK, 128), jnp.inf, jnp.float32)
    pc0 = jnp.full((RBLK, 128), BIGI, jnp.int32)
    pv, pc = lax.fori_loop(0, NCH, build, (pv0, pc0))

    res = jnp.zeros((RBLK, K), jnp.int32)
    cnt = jnp.zeros((RBLK, 128), jnp.int32)
    for t in range(K):
        cm = jnp.min(pv, axis=1, keepdims=True)
        amc = jnp.min(jnp.where(pv == cm, pc, BIGI), axis=1, keepdims=True)
        res = jnp.where(kcol == t, amc, res)
        pv = jnp.where(pc == amc, jnp.inf, pv)
        cnt = cnt + jnp.where(piota == amc // CW, 1, 0)
    anybad = jnp.max(cnt) >= DEPTH

    def fallback(res_):
        def sel(t, carry):
            res, am_prev = carry

            def scan_chunk(c, mcarry):
                m, am = mcarry
                off = pl.multiple_of(c * CW, CW)
                chunk = d2_ref[:, pl.ds(off, CW)]
                colc = citer + off
                chunk = jnp.where(colc == am_prev, jnp.inf, chunk)
                d2_ref[:, pl.ds(off, CW)] = chunk
                cm = jnp.min(chunk, axis=1, keepdims=True)
                cam = jnp.min(jnp.where(chunk <= cm, colc, PAD),
                              axis=1, keepdims=True)
                take = (cm < m) | ((cm == m) & (cam < am))
                return jnp.where(take, cm, m), jnp.where(take, cam, am)

            m0 = jnp.full((RBLK, 1), jnp.inf, jnp.float32)
            am0 = jnp.full((RBLK, 1), PAD, jnp.int32)
            m, am = lax.fori_loop(0, NCH, scan_chunk, (m0, am0))
            return jnp.where(kcol == t, am, res), am

        res0 = jnp.zeros((RBLK, K), jnp.int32)
        amp0 = jnp.full((RBLK, 1), -1, jnp.int32)
        out, _ = lax.fori_loop(0, K, sel, (res0, amp0))
        return out

    res = lax.cond(anybad, fallback, lambda r: r, res)
    idx_ref[...] = res


def _knn(pos8, post8):
    return pl.pallas_call(
        _knn_body,
        grid=(PAD // RBLK,),
        in_specs=[
            pl.BlockSpec((RBLK, 8), lambda i: (i, 0)),
            pl.BlockSpec((8, PAD), lambda i: (0, 0)),
        ],
        out_specs=pl.BlockSpec((RBLK, K), lambda i: (i, 0)),
        out_shape=jax.ShapeDtypeStruct((PAD, K), jnp.int32),
        scratch_shapes=[pltpu.VMEM((RBLK, PAD), jnp.float32)],
    )(pos8, post8)


def _mm_body(x_ref, w_ref, b_ref, a_ref, g_ref, *, c_in):
    x = x_ref[...]
    w = w_ref[...]
    wt = w[0:c_in, :]
    wb = w[c_in:2 * c_in, :]
    g_ref[...] = jnp.dot(x, wb, preferred_element_type=jnp.float32)
    a_ref[...] = jnp.dot(x, wt - wb, preferred_element_type=jnp.float32) \
        + b_ref[...]


MMB = 1024


def _mm(xp, w, b2d, c_in, c_out):
    return pl.pallas_call(
        functools.partial(_mm_body, c_in=c_in),
        grid=(PAD // MMB,),
        in_specs=[
            pl.BlockSpec((MMB, c_in), lambda i: (i, 0)),
            pl.BlockSpec((2 * c_in, c_out), lambda i: (0, 0)),
            pl.BlockSpec((1, c_out), lambda i: (0, 0)),
        ],
        out_specs=[pl.BlockSpec((MMB, c_out), lambda i: (i, 0)),
                   pl.BlockSpec((MMB, c_out), lambda i: (i, 0))],
        out_shape=[jax.ShapeDtypeStruct((PAD, c_out), jnp.float32),
                   jax.ShapeDtypeStruct((PAD, c_out), jnp.float32)],
    )(xp, w, b2d)


@functools.cache
def _make_gather_max(c_out):
    nseg = c_out // 16
    mesh = plsc.VectorSubcoreMesh(core_axis_name="c", subcore_axis_name="s")

    @functools.partial(
        pl.kernel, mesh=mesh,
        out_type=jax.ShapeDtypeStruct((PAD, c_out), jnp.float32),
        scratch_types=[
            pltpu.VMEM((EDGES_PER_CHUNK,), jnp.int32),
            pltpu.VMEM((EDGES_PER_CHUNK, c_out), jnp.float32),
            pltpu.VMEM((ROWS_PER_W, c_out), jnp.float32),
            pltpu.VMEM((ROWS_PER_W, c_out), jnp.float32),
            pltpu.SemaphoreType.DMA,
        ],
        compiler_params=pltpu.CompilerParams(use_tc_tiling_on_sc=False),
    )
    def gather_max(idx_hbm, g_hbm, a_hbm, out_hbm,
                   idx_v, rows_v, a_v, out_v, sem):
        wid = lax.axis_index("s") * 2 + lax.axis_index("c")
        base = wid * ROWS_PER_W
        pltpu.sync_copy(a_hbm.at[pl.ds(base, ROWS_PER_W)], a_v)

        def chunk(kk, carry):
            ebase = base * K + kk * EDGES_PER_CHUNK
            pltpu.sync_copy(idx_hbm.at[pl.ds(ebase, EDGES_PER_CHUNK)], idx_v)
            pltpu.async_copy(g_hbm.at[idx_v], rows_v, sem).wait()

            def row(r, c2):
                e0 = r * K
                orow = kk * CHUNK_ROWS + r
                for s in range(nseg):
                    sl = pl.ds(s * 16, 16)
                    acc = rows_v[e0, sl]
                    for j in range(1, K):
                        acc = jnp.maximum(acc, rows_v[e0 + j, sl])
                    out_v[orow, sl] = jnp.maximum(acc + a_v[orow, sl], 0.0)
                return c2

            lax.fori_loop(0, CHUNK_ROWS, row, 0)
            return carry

        lax.fori_loop(0, NCHUNK, chunk, 0)
        pltpu.sync_copy(out_v, out_hbm.at[pl.ds(base, ROWS_PER_W)])

    return gather_max


def kernel(point_coords, point_features, W0, b0, W1, b1, W2, b2):
    pos = point_coords[:, 1:4]
    pos8 = jnp.zeros((PAD, 8), jnp.float32).at[:N, :3].set(pos)
    post8 = pos8.T
    idx_flat = _knn(pos8, post8).reshape(PAD * K)

    xp = jnp.zeros((PAD, point_features.shape[1]), jnp.float32)
    xp = xp.at[:N].set(point_features)
    for w, b in ((W0, b0), (W1, b1), (W2, b2)):
        c_in, c_out = w.shape[0] // 2, w.shape[1]
        a, g = _mm(xp, w, b.reshape(1, c_out), c_in, c_out)
        xp = _make_gather_max(c_out)(idx_flat, g, a)
    return xp[:N]


# speedup vs baseline: 4.0179x; 1.0268x over previous
import functools

import jax
import jax.numpy as jnp
from jax import lax
from jax.experimental import pallas as pl
from jax.experimental.pallas import tpu as pltpu
from jax.experimental.pallas import tpu_sc as plsc

N = 10000
K = 16
PAD = 10240
RBLK = 1024
NW = 32
ROWS_PER_W = PAD // NW
CHUNK_ROWS = 8
EDGES_PER_CHUNK = CHUNK_ROWS * K
NCHUNK = ROWS_PER_W // CHUNK_ROWS


CW = 640
NCH = PAD // CW
DEPTH = 8
POOL = NCH * DEPTH
BIGI = 1 << 30


def _knn_body(pos_ref, post_ref, idx_ref, d2_ref):
    pos = pos_ref[...]
    sqi = (pos[:, 0:1] * pos[:, 0:1] + pos[:, 1:2] * pos[:, 1:2]) \
        + pos[:, 2:3] * pos[:, 2:3]
    citer = lax.broadcasted_iota(jnp.int32, (RBLK, CW), 1)
    piota = lax.broadcasted_iota(jnp.int32, (RBLK, 128), 1)
    kcol = lax.broadcasted_iota(jnp.int32, (RBLK, K), 1)

    def build(c, carry):
        pv, pc = carry
        off = pl.multiple_of(c * CW, CW)
        postc = post_ref[:, pl.ds(off, CW)]
        dotc = jnp.dot(pos, postc, preferred_element_type=jnp.float32)
        sqjc = (postc[0:1, :] * postc[0:1, :]
                + postc[1:2, :] * postc[1:2, :]) \
            + postc[2:3, :] * postc[2:3, :]
        colc = citer + off
        d2c = sqi + sqjc - 2.0 * dotc
        d2c = jnp.where(colc >= N, jnp.inf, d2c)
        d2_ref[:, pl.ds(off, CW)] = d2c
        for j in range(DEPTH):
            cm = jnp.min(d2c, axis=1, keepdims=True)
            cam = jnp.min(jnp.where(d2c == cm, colc, BIGI),
                          axis=1, keepdims=True)
            d2c = jnp.where(colc == cam, jnp.inf, d2c)
            lane = c * DEPTH + j
            pv = jnp.where(piota == lane, cm, pv)
            pc = jnp.where(piota == lane, cam, pc)
        return pv, pc

    pv0 = jnp.full((RBLK, 128), jnp.inf, jnp.float32)
    pc0 = jnp.full((RBLK, 128), BIGI, jnp.int32)
    pv, pc = lax.fori_loop(0, NCH, build, (pv0, pc0))

    res = jnp.zeros((RBLK, K), jnp.int32)
    cnt = jnp.zeros((RBLK, 128), jnp.int32)
    for t in range(K):
        cm = jnp.min(pv, axis=1, keepdims=True)
        amc = jnp.min(jnp.where(pv == cm, pc, BIGI), axis=1, keepdims=True)
        res = jnp.where(kcol == t, amc, res)
        pv = jnp.where(pc == amc, jnp.inf, pv)
        cnt = cnt + jnp.where(piota == amc // CW, 1, 0)
    anybad = jnp.max(cnt) >= DEPTH

    def fallback(res_):
        def sel(t, carry):
            res, am_prev = carry

            def scan_chunk(c, mcarry):
                m, am = mcarry
                off = pl.multiple_of(c * CW, CW)
                chunk = d2_ref[:, pl.ds(off, CW)]
                colc = citer + off
                chunk = jnp.where(colc == am_prev, jnp.inf, chunk)
                d2_ref[:, pl.ds(off, CW)] = chunk
                cm = jnp.min(chunk, axis=1, keepdims=True)
                cam = jnp.min(jnp.where(chunk <= cm, colc, PAD),
                              axis=1, keepdims=True)
                take = (cm < m) | ((cm == m) & (cam < am))
                return jnp.where(take, cm, m), jnp.where(take, cam, am)

            m0 = jnp.full((RBLK, 1), jnp.inf, jnp.float32)
            am0 = jnp.full((RBLK, 1), PAD, jnp.int32)
            m, am = lax.fori_loop(0, NCH, scan_chunk, (m0, am0))
            return jnp.where(kcol == t, am, res), am

        res0 = jnp.zeros((RBLK, K), jnp.int32)
        amp0 = jnp.full((RBLK, 1), -1, jnp.int32)
        out, _ = lax.fori_loop(0, K, sel, (res0, amp0))
        return out

    res = lax.cond(anybad, fallback, lambda r: r, res)
    idx_ref[...] = res


def _knn(pos8, post8):
    return pl.pallas_call(
        _knn_body,
        grid=(PAD // RBLK,),
        in_specs=[
            pl.BlockSpec((RBLK, 8), lambda i: (i, 0)),
            pl.BlockSpec((8, PAD), lambda i: (0, 0)),
        ],
        out_specs=pl.BlockSpec((RBLK, K), lambda i: (i, 0)),
        out_shape=jax.ShapeDtypeStruct((PAD, K), jnp.int32),
        scratch_shapes=[pltpu.VMEM((RBLK, PAD), jnp.float32)],
    )(pos8, post8)


def _mm_body(x_ref, w_ref, b_ref, a_ref, g_ref, *, c_in):
    x = x_ref[...]
    w = w_ref[...]
    wt = w[0:c_in, :]
    wb = w[c_in:2 * c_in, :]
    g_ref[...] = jnp.dot(x, wb, preferred_element_type=jnp.float32)
    a_ref[...] = jnp.dot(x, wt - wb, preferred_element_type=jnp.float32) \
        + b_ref[...]


MMB = 1024


def _mm(xp, w, b2d, c_in, c_out):
    return pl.pallas_call(
        functools.partial(_mm_body, c_in=c_in),
        grid=(PAD // MMB,),
        in_specs=[
            pl.BlockSpec((MMB, c_in), lambda i: (i, 0)),
            pl.BlockSpec((2 * c_in, c_out), lambda i: (0, 0)),
            pl.BlockSpec((1, c_out), lambda i: (0, 0)),
        ],
        out_specs=[pl.BlockSpec((MMB, c_out), lambda i: (i, 0)),
                   pl.BlockSpec((MMB, c_out), lambda i: (i, 0))],
        out_shape=[jax.ShapeDtypeStruct((PAD, c_out), jnp.float32),
                   jax.ShapeDtypeStruct((PAD, c_out), jnp.float32)],
    )(xp, w, b2d)


@functools.cache
def _make_gather_max(c_out):
    nseg = c_out // 16
    mesh = plsc.VectorSubcoreMesh(core_axis_name="c", subcore_axis_name="s")

    @functools.partial(
        pl.kernel, mesh=mesh,
        out_type=jax.ShapeDtypeStruct((PAD, c_out), jnp.float32),
        scratch_types=[
            pltpu.VMEM((EDGES_PER_CHUNK,), jnp.int32),
            pltpu.VMEM((EDGES_PER_CHUNK, c_out), jnp.float32),
            pltpu.VMEM((ROWS_PER_W, c_out), jnp.float32),
            pltpu.VMEM((ROWS_PER_W, c_out), jnp.float32),
            pltpu.SemaphoreType.DMA,
        ],
        compiler_params=pltpu.CompilerParams(use_tc_tiling_on_sc=False),
    )
    def gather_max(idx_hbm, g_hbm, a_hbm, out_hbm,
                   idx_v, rows_v, a_v, out_v, sem):
        wid = lax.axis_index("s") * 2 + lax.axis_index("c")
        base = wid * ROWS_PER_W
        pltpu.sync_copy(a_hbm.at[pl.ds(base, ROWS_PER_W)], a_v)

        def chunk(kk, carry):
            ebase = base * K + kk * EDGES_PER_CHUNK
            pltpu.sync_copy(idx_hbm.at[pl.ds(ebase, EDGES_PER_CHUNK)], idx_v)
            pltpu.async_copy(g_hbm.at[idx_v], rows_v, sem).wait()

            def row(r, c2):
                e0 = r * K
                orow = kk * CHUNK_ROWS + r
                for s in range(nseg):
                    sl = pl.ds(s * 16, 16)
                    acc = rows_v[e0, sl]
                    for j in range(1, K):
                        acc = jnp.maximum(acc, rows_v[e0 + j, sl])
                    out_v[orow, sl] = jnp.maximum(acc + a_v[orow, sl], 0.0)
                return c2

            lax.fori_loop(0, CHUNK_ROWS, row, 0)
            return carry

        lax.fori_loop(0, NCHUNK, chunk, 0)
        pltpu.sync_copy(out_v, out_hbm.at[pl.ds(base, ROWS_PER_W)])

    return gather_max


def kernel(point_coords, point_features, W0, b0, W1, b1, W2, b2):
    pos = point_coords[:, 1:4]
    pos8 = jnp.zeros((PAD, 8), jnp.float32).at[:N, :3].set(pos)
    post8 = pos8.T
    idx_flat = _knn(pos8, post8).reshape(PAD * K)

    xp = jnp.zeros((PAD, point_features.shape[1]), jnp.float32)
    xp = xp.at[:N].set(point_features)
    for w, b in ((W0, b0), (W1, b1), (W2, b2)):
        c_in, c_out = w.shape[0] // 2, w.shape[1]
        a, g = _mm(xp, w, b.reshape(1, c_out), c_in, c_out)
        xp = _make_gather_max(c_out)(idx_flat, g, a)
    return xp[:N]
